# baseline (speedup 1.0000x reference)
import functools

import jax
import jax.numpy as jnp
from jax.experimental import pallas as pl
from jax.experimental.pallas import tpu as pltpu

K_SEL = 64
T_DEPTH = 12
NLANE = 128
O_CLS = 21
O_PAD = 24
HS = 8
HT = 16
N_DMA = 16
PACKED = T_DEPTH // 2 * NLANE


def _keys(x):
    raw = jax.lax.bitcast_convert_type(x, jnp.int32)
    return jnp.where(raw >= 0, raw, raw ^ jnp.int32(0x7FFFFFFF))


def _unkey(acc):
    e_raw = jnp.where(acc >= 0, acc, acc ^ jnp.int32(0x7FFFFFFF))
    return jax.lax.bitcast_convert_type(e_raw, jnp.float32)


def _search_64th(key):
    kd = dict(axis=1, keepdims=True)
    c0 = jnp.sum((key >= 0).astype(jnp.int32), **kd)
    acc = jnp.where(c0 >= K_SEL, jnp.int32(0), jnp.int32(-2147483648))

    def bit_step(i, acc):
        bit = 30 - i
        cand = acc | (jnp.int32(1) << bit)
        c = jnp.sum((key >= cand).astype(jnp.int32), **kd)
        return jnp.where(c >= K_SEL, cand, acc)

    return jax.lax.fori_loop(0, 31, bit_step, acc)


def _issue_copies(feat_hbm, buf, sems, b, slot):
    csz = 96 // N_DMA
    for k in range(N_DMA):
        pltpu.make_async_copy(
            feat_hbm.at[b, k * csz:(k + 1) * csz],
            buf.at[slot, k * csz:(k + 1) * csz],
            sems.at[slot, k],
        ).start()


def _wait_copies(feat_hbm, buf, sems, b, slot):
    csz = 96 // N_DMA
    for k in range(N_DMA):
        pltpu.make_async_copy(
            feat_hbm.at[b, k * csz:(k + 1) * csz],
            buf.at[slot, k * csz:(k + 1) * csz],
            sems.at[slot, k],
        ).wait()


def _fused_body(feat_hbm, w_ref, bias_ref, out_ref, flag_ref, buf, sems,
                acc_ref):
    w192 = w_ref[...]
    b = pl.program_id(0)
    nb = pl.num_programs(0)
    slot = jax.lax.rem(b, 2)

    @pl.when(b == 0)
    def _():
        _issue_copies(feat_hbm, buf, sems, 0, 0)

    @pl.when(b + 1 < nb)
    def _():
        _issue_copies(feat_hbm, buf, sems, b + 1, 1 - slot)

    _wait_copies(feat_hbm, buf, sems, b, slot)

    neg_inf = jnp.float32(float("-inf"))
    T = [jnp.full((O_PAD, NLANE), neg_inf, jnp.float32)
         for _ in range(T_DEPTH)]
    for ht in range(HT):
        rhs = buf[slot, :, ht].reshape(HS * 96, NLANE)
        res = jax.lax.dot_general(
            w192, rhs, (((1,), (0,)), ((), ())),
            preferred_element_type=jnp.float32,
        )
        for hs in range(HS):
            c = res[hs * O_PAD:(hs + 1) * O_PAD, :]
            for t in range(T_DEPTH):
                hi = jnp.maximum(T[t], c)
                c = jnp.minimum(T[t], c)
                T[t] = hi

    Trot = [jnp.concatenate([t[:, 64:], t[:, :64]], axis=1) for t in T]
    M = [jnp.maximum(T[i], Trot[T_DEPTH - 1 - i]) for i in range(T_DEPTH)]
    M = M + [jnp.full((O_PAD, NLANE), neg_inf, jnp.float32)] * 4
    k = 2
    while k <= 16:
        j = k // 2
        while j >= 1:
            for i in range(16):
                l = i ^ j
                if l > i:
                    hi = jnp.maximum(M[i], M[l])
                    lo = jnp.minimum(M[i], M[l])
                    if (i & k) == 0:
                        M[i], M[l] = hi, lo
                    else:
                        M[i], M[l] = lo, hi
            j //= 2
        k *= 2
    P = [jnp.concatenate([M[2 * k][:, :64], M[2 * k + 1][:, :64]], axis=1)
         for k in range(T_DEPTH // 2)]
    acc_ref[pl.ds(b * O_PAD, O_PAD), :] = jnp.concatenate(P, axis=1)

    @pl.when(b == nb - 1)
    def _():
        cand = acc_ref[...]
        ckey = _keys(cand)
        acc = _search_64th(ckey)
        tstar = _unkey(acc)

        gt = ckey > acc
        cgt = jnp.sum(gt.astype(jnp.int32), axis=1, keepdims=True)
        s = jnp.sum(jnp.where(gt, cand, 0.0), axis=1, keepdims=True)
        res = (s + (K_SEL - cgt).astype(jnp.float32) * tstar) / K_SEL
        out_ref[...] = res + bias_ref[...]

        floor_lvl = cand[:, (T_DEPTH // 2 - 1) * NLANE + 64:
                         (T_DEPTH // 2) * NLANE]
        pair_floor = jnp.max(floor_lvl, axis=1, keepdims=True)
        ok = jnp.all(tstar >= pair_floor)
        flag_ref[...] = ok.astype(jnp.int32).reshape(1, 1)


def _exact_body(feat_ref, w_ref, b_ref, out_ref):
    f = feat_ref[0]
    w = w_ref[...]
    logits = jax.lax.dot_general(
        w, f, (((1,), (0,)), ((), ())),
        preferred_element_type=jnp.float32,
    )
    logits = logits + b_ref[0][:, None]
    key = _keys(logits)
    acc = _search_64th(key)
    gt = key > acc
    cgt = jnp.sum(gt.astype(jnp.int32), axis=1, keepdims=True)
    s = jnp.sum(jnp.where(gt, logits, 0.0), axis=1, keepdims=True)
    res = (s + (K_SEL - cgt).astype(jnp.float32) * _unkey(acc)) / K_SEL
    out_ref[...] = res[None]


def _exact_path(feat, W, b):
    B, C, H, Wd = feat.shape
    featr = feat.reshape(B, C, H * Wd)
    out = pl.pallas_call(
        _exact_body,
        grid=(B,),
        in_specs=[
            pl.BlockSpec((1, C, H * Wd), lambda i: (i, 0, 0)),
            pl.BlockSpec((O_CLS, C), lambda i: (0, 0)),
            pl.BlockSpec((1, O_CLS), lambda i: (0, 0)),
        ],
        out_specs=pl.BlockSpec((1, O_CLS, 1), lambda i: (i, 0, 0)),
        out_shape=jax.ShapeDtypeStruct((B, O_CLS, 1), jnp.float32),
    )(featr, W, b[None, :])
    return out.reshape(B, O_CLS)


@functools.partial(jax.jit, static_argnames=())
def kernel(feat, W, b):
    B, C, H, Wd = feat.shape
    feat5 = feat.reshape(B, C, HT, HS, Wd)

    e8 = jnp.eye(HS, dtype=W.dtype)
    w4 = W[None, :, :, None] * e8[:, None, None, :]
    w4 = jnp.pad(w4, ((0, 0), (0, O_PAD - O_CLS), (0, 0), (0, 0)))
    w192 = w4.reshape(HS * O_PAD, C * HS)

    rows = B * O_PAD
    bias = jnp.tile(jnp.pad(b, (0, O_PAD - O_CLS)), B).reshape(rows, 1)

    res, flag = pl.pallas_call(
        _fused_body,
        grid=(B,),
        in_specs=[
            pl.BlockSpec(memory_space=pl.ANY),
            pl.BlockSpec((HS * O_PAD, C * HS), lambda i: (0, 0)),
            pl.BlockSpec((rows, 1), lambda i: (0, 0)),
        ],
        out_specs=[
            pl.BlockSpec((rows, 1), lambda i: (0, 0)),
            pl.BlockSpec((1, 1), lambda i: (0, 0)),
        ],
        out_shape=[
            jax.ShapeDtypeStruct((rows, 1), jnp.float32),
            jax.ShapeDtypeStruct((1, 1), jnp.int32),
        ],
        scratch_shapes=[
            pltpu.VMEM((2, C, HT, HS, Wd), jnp.float32),
            pltpu.SemaphoreType.DMA((2, N_DMA)),
            pltpu.VMEM((rows, PACKED), jnp.float32),
        ],
    )(feat5, w192, bias)

    fast = res.reshape(B, O_PAD)[:, :O_CLS]
    logits = jax.lax.cond(
        flag[0, 0] > 0,
        lambda: fast,
        lambda: _exact_path(feat, W, b),
    )
    bs_loss = jnp.zeros((), dtype=jnp.float32)
    return (logits, bs_loss)


# speedup vs baseline: 1.0496x; 1.0496x over previous
import functools

import jax
import jax.numpy as jnp
from jax.experimental import pallas as pl
from jax.experimental.pallas import tpu as pltpu

K_SEL = 64
T_DEPTH = 10
NLANE = 128
O_CLS = 21
O_PAD = 24
HS = 8
HT = 16
N_DMA = 8
PACKED = T_DEPTH // 2 * NLANE


def _keys(x):
    raw = jax.lax.bitcast_convert_type(x, jnp.int32)
    return jnp.where(raw >= 0, raw, raw ^ jnp.int32(0x7FFFFFFF))


def _unkey(acc):
    e_raw = jnp.where(acc >= 0, acc, acc ^ jnp.int32(0x7FFFFFFF))
    return jax.lax.bitcast_convert_type(e_raw, jnp.float32)


def _search_64th(key):
    kd = dict(axis=1, keepdims=True)
    c0 = jnp.sum((key >= 0).astype(jnp.int32), **kd)
    acc = jnp.where(c0 >= K_SEL, jnp.int32(0), jnp.int32(-2147483648))

    def bit_step(i, acc):
        bit = 30 - i
        cand = acc | (jnp.int32(1) << bit)
        c = jnp.sum((key >= cand).astype(jnp.int32), **kd)
        return jnp.where(c >= K_SEL, cand, acc)

    return jax.lax.fori_loop(0, 31, bit_step, acc)


def _issue_copies(feat_hbm, buf, sems, b, slot):
    csz = 96 // N_DMA
    for k in range(N_DMA):
        pltpu.make_async_copy(
            feat_hbm.at[b, k * csz:(k + 1) * csz],
            buf.at[slot, k * csz:(k + 1) * csz],
            sems.at[slot, k],
        ).start()


def _wait_copies(feat_hbm, buf, sems, b, slot):
    csz = 96 // N_DMA
    for k in range(N_DMA):
        pltpu.make_async_copy(
            feat_hbm.at[b, k * csz:(k + 1) * csz],
            buf.at[slot, k * csz:(k + 1) * csz],
            sems.at[slot, k],
        ).wait()


def _fused_body(feat_hbm, w_ref, bias_ref, out_ref, flag_ref, buf, sems,
                acc_ref):
    w192 = w_ref[...]
    b = pl.program_id(0)
    nb = pl.num_programs(0)
    slot = jax.lax.rem(b, 2)

    @pl.when(b == 0)
    def _():
        _issue_copies(feat_hbm, buf, sems, 0, 0)

    @pl.when(b + 1 < nb)
    def _():
        _issue_copies(feat_hbm, buf, sems, b + 1, 1 - slot)

    _wait_copies(feat_hbm, buf, sems, b, slot)

    neg_inf = jnp.float32(float("-inf"))
    T = [jnp.full((O_PAD, NLANE), neg_inf, jnp.float32)
         for _ in range(T_DEPTH)]
    for ht in range(HT):
        rhs = buf[slot, :, ht].reshape(HS * 96, NLANE)
        res = jax.lax.dot_general(
            w192, rhs, (((1,), (0,)), ((), ())),
            preferred_element_type=jnp.float32,
        )
        for hs in range(HS):
            c = res[hs * O_PAD:(hs + 1) * O_PAD, :]
            for t in range(T_DEPTH):
                hi = jnp.maximum(T[t], c)
                c = jnp.minimum(T[t], c)
                T[t] = hi

    Trot = [jnp.concatenate([t[:, 64:], t[:, :64]], axis=1) for t in T]
    M = [jnp.maximum(T[i], Trot[T_DEPTH - 1 - i]) for i in range(T_DEPTH)]
    M = M + [jnp.full((O_PAD, NLANE), neg_inf, jnp.float32)] * (16 - T_DEPTH)
    k = 2
    while k <= 16:
        j = k // 2
        while j >= 1:
            for i in range(16):
                l = i ^ j
                if l > i:
                    hi = jnp.maximum(M[i], M[l])
                    lo = jnp.minimum(M[i], M[l])
                    if (i & k) == 0:
                        M[i], M[l] = hi, lo
                    else:
                        M[i], M[l] = lo, hi
            j //= 2
        k *= 2
    P = [jnp.concatenate([M[2 * k][:, :64], M[2 * k + 1][:, :64]], axis=1)
         for k in range(T_DEPTH // 2)]
    acc_ref[pl.ds(b * O_PAD, O_PAD), :] = jnp.concatenate(P, axis=1)

    @pl.when(b == nb - 1)
    def _():
        cand = acc_ref[...]
        ckey = _keys(cand)
        acc = _search_64th(ckey)
        tstar = _unkey(acc)

        gt = ckey > acc
        cgt = jnp.sum(gt.astype(jnp.int32), axis=1, keepdims=True)
        s = jnp.sum(jnp.where(gt, cand, 0.0), axis=1, keepdims=True)
        res = (s + (K_SEL - cgt).astype(jnp.float32) * tstar) / K_SEL
        out_ref[...] = res + bias_ref[...]

        floor_lvl = cand[:, (T_DEPTH // 2 - 1) * NLANE + 64:
                         (T_DEPTH // 2) * NLANE]
        pair_floor = jnp.max(floor_lvl, axis=1, keepdims=True)
        ok = jnp.all(tstar >= pair_floor)
        flag_ref[...] = ok.astype(jnp.int32).reshape(1, 1)


def _exact_body(feat_ref, w_ref, b_ref, out_ref):
    f = feat_ref[0]
    w = w_ref[...]
    logits = jax.lax.dot_general(
        w, f, (((1,), (0,)), ((), ())),
        preferred_element_type=jnp.float32,
    )
    logits = logits + b_ref[0][:, None]
    key = _keys(logits)
    acc = _search_64th(key)
    gt = key > acc
    cgt = jnp.sum(gt.astype(jnp.int32), axis=1, keepdims=True)
    s = jnp.sum(jnp.where(gt, logits, 0.0), axis=1, keepdims=True)
    res = (s + (K_SEL - cgt).astype(jnp.float32) * _unkey(acc)) / K_SEL
    out_ref[...] = res[None]


def _exact_path(feat, W, b):
    B, C, H, Wd = feat.shape
    featr = feat.reshape(B, C, H * Wd)
    out = pl.pallas_call(
        _exact_body,
        grid=(B,),
        in_specs=[
            pl.BlockSpec((1, C, H * Wd), lambda i: (i, 0, 0)),
            pl.BlockSpec((O_CLS, C), lambda i: (0, 0)),
            pl.BlockSpec((1, O_CLS), lambda i: (0, 0)),
        ],
        out_specs=pl.BlockSpec((1, O_CLS, 1), lambda i: (i, 0, 0)),
        out_shape=jax.ShapeDtypeStruct((B, O_CLS, 1), jnp.float32),
    )(featr, W, b[None, :])
    return out.reshape(B, O_CLS)


@functools.partial(jax.jit, static_argnames=())
def kernel(feat, W, b):
    B, C, H, Wd = feat.shape
    feat5 = feat.reshape(B, C, HT, HS, Wd)

    e8 = jnp.eye(HS, dtype=W.dtype)
    w4 = W[None, :, :, None] * e8[:, None, None, :]
    w4 = jnp.pad(w4, ((0, 0), (0, O_PAD - O_CLS), (0, 0), (0, 0)))
    w192 = w4.reshape(HS * O_PAD, C * HS)

    rows = B * O_PAD
    bias = jnp.tile(jnp.pad(b, (0, O_PAD - O_CLS)), B).reshape(rows, 1)

    res, flag = pl.pallas_call(
        _fused_body,
        grid=(B,),
        in_specs=[
            pl.BlockSpec(memory_space=pl.ANY),
            pl.BlockSpec((HS * O_PAD, C * HS), lambda i: (0, 0)),
            pl.BlockSpec((rows, 1), lambda i: (0, 0)),
        ],
        out_specs=[
            pl.BlockSpec((rows, 1), lambda i: (0, 0)),
            pl.BlockSpec((1, 1), lambda i: (0, 0)),
        ],
        out_shape=[
            jax.ShapeDtypeStruct((rows, 1), jnp.float32),
            jax.ShapeDtypeStruct((1, 1), jnp.int32),
        ],
        scratch_shapes=[
            pltpu.VMEM((2, C, HT, HS, Wd), jnp.float32),
            pltpu.SemaphoreType.DMA((2, N_DMA)),
            pltpu.VMEM((rows, PACKED), jnp.float32),
        ],
    )(feat5, w192, bias)

    fast = res.reshape(B, O_PAD)[:, :O_CLS]
    logits = jax.lax.cond(
        flag[0, 0] > 0,
        lambda: fast,
        lambda: _exact_path(feat, W, b),
    )
    bs_loss = jnp.zeros((), dtype=jnp.float32)
    return (logits, bs_loss)
